# Initial kernel scaffold; baseline (speedup 1.0000x reference)
import jax
import jax.numpy as jnp
from jax.experimental import pallas as pl


def kernel(x, edge_index, edge_attr, W1, R1, B1, W2, R2, B2, W3, R3, B3, W4, R4, B4, W5, R5, B5, W6, R6, B6):
    raise NotImplementedError("write your pallas kernel here")


# speedup vs baseline: 5.1721x; 5.1721x over previous
import functools

import jax
import jax.numpy as jnp
from jax import lax
from jax.experimental import pallas as pl
from jax.experimental.pallas import tpu as pltpu
from jax.experimental.pallas import tpu_sc as plsc

NC = 2
NS = 16
NW = NC * NS
C = 80
CPB = 25
F32 = jnp.float32
I32 = jnp.int32


def _mm_body(h_ref, w_ref, o_ref):
    o_ref[...] = jnp.dot(h_ref[...], w_ref[...], preferred_element_type=F32)


def _matmul(h, w, bn=1000):
    n, cin = h.shape
    m = w.shape[1]
    return pl.pallas_call(
        _mm_body,
        grid=(n // bn,),
        in_specs=[pl.BlockSpec((bn, cin), lambda i: (i, 0)),
                  pl.BlockSpec((cin, m), lambda i: (0, 0))],
        out_specs=pl.BlockSpec((bn, m), lambda i: (i, 0)),
        out_shape=jax.ShapeDtypeStruct((n, m), F32),
    )(h, w)


_KS = (5, 7, 11)


def _prep_body(s_ref, u_ref, *o_refs):
    s = s_ref[...]
    u = jnp.clip(u_ref[...], 0.0, 1.0)
    i = 0
    for k in _KS:
        v = u * (k - 1.0)
        f0 = jnp.floor(v)
        i0 = jnp.clip(f0.astype(I32), 0, k - 1)
        frac = v - f0
        o_refs[i][...] = s * k + i0
        o_refs[i + 1][...] = 1.0 - frac
        o_refs[i + 2][...] = frac
        i += 3


def _prep(src2, u2, bn=1000):
    r = src2.shape[0]
    spec = pl.BlockSpec((bn, 128), lambda i: (i, 0))
    shapes = []
    for _ in _KS:
        shapes += [jax.ShapeDtypeStruct((r, 128), I32)]
        shapes += [jax.ShapeDtypeStruct((r, 128), F32)] * 2
    return pl.pallas_call(
        _prep_body,
        grid=(r // bn,),
        in_specs=[spec, spec],
        out_specs=[spec] * 9,
        out_shape=shapes,
    )(src2, u2)


@functools.lru_cache(maxsize=None)
def _edge_pass(n, e, tw, cw, acw):
    epw = e // NW
    nblk = epw // (C * CPB)
    rt = (n // NS) // 8 * 8
    tail = n - rt * NS
    zb = 16
    ncol = cw // 16
    mesh = plsc.VectorSubcoreMesh(core_axis_name="c", subcore_axis_name="s",
                                  num_cores=NC, num_subcores=NS)

    def body(t, g0v, w0, w1, dstv, out,
             g0b, dstb, w0b, w1b, rows, comb, zbuf, acc, sem):
        cid = lax.axis_index("c")
        sid = lax.axis_index("s")

        def zfill(i, _):
            for tt in range(acw // 16):
                zbuf[i, pl.ds(tt * 16, 16)] = jnp.zeros((16,), F32)
            return 0
        lax.fori_loop(0, zb, zfill, 0)

        if cw < acw:
            def cfill(i, _):
                for tt in range(cw // 16, acw // 16):
                    comb[i, pl.ds(tt * 16, 16)] = jnp.zeros((16,), F32)
                return 0
            lax.fori_loop(0, C, cfill, 0)
        rbase = sid * rt
        for j in range(rt // zb):
            pltpu.sync_copy(zbuf, acc.at[pl.ds(rbase + j * zb, zb)])

        @pl.when(sid == 0)
        def _():
            pltpu.sync_copy(zbuf.at[pl.ds(0, tail)],
                            acc.at[pl.ds(NS * rt, tail)])
        plsc.subcore_barrier()

        wid = cid * NS + sid
        ebase = wid * epw

        def block(nb, _):
            eb = ebase + nb * (C * CPB)
            pltpu.sync_copy(g0v.at[wid * nblk + nb], g0b)
            pltpu.sync_copy(dstv.at[wid * nblk + nb], dstb)
            pltpu.sync_copy(w0.at[pl.ds(eb, C * CPB)], w0b)
            pltpu.sync_copy(w1.at[pl.ds(eb, C * CPB)], w1b)

            def chunk(c, _):
                cb = c * C
                pltpu.async_copy(t.at[g0b.at[c]], rows, sem).wait()

                def group(g, _):
                    w0g = w0b[pl.ds(cb + g * 16, 16)]
                    w1g = w1b[pl.ds(cb + g * 16, 16)]
                    for j2 in range(16):
                        jj = g * 16 + j2
                        sel = jnp.full((16,), j2, I32)
                        a = w0g.at[sel].get(mode="promise_in_bounds")
                        b = w1g.at[sel].get(mode="promise_in_bounds")
                        for tt in range(ncol):
                            sl = pl.ds(tt * 16, 16)
                            comb[jj, sl] = (
                                rows[jj, sl] * a
                                + rows[jj, pl.ds(cw + tt * 16, 16)] * b)
                    return 0
                lax.fori_loop(0, C // 16, group, 0)
                pltpu.sync_copy(comb, acc.at[dstb.at[c]], add=True)
                return 0
            lax.fori_loop(0, CPB, chunk, 0)
            return 0
        lax.fori_loop(0, nblk, block, 0)
        plsc.subcore_barrier()

        pltpu.sync_copy(acc.at[pl.ds(rbase, rt)],
                        out.at[cid, pl.ds(rbase, rt)])

        @pl.when(sid == 0)
        def _():
            pltpu.sync_copy(acc.at[pl.ds(NS * rt, tail)],
                            out.at[cid, pl.ds(NS * rt, tail)])

    return pl.kernel(
        body,
        out_type=jax.ShapeDtypeStruct((NC, n, acw), F32),
        mesh=mesh,
        scratch_types=[
            pltpu.VMEM((CPB, C), I32),
            pltpu.VMEM((CPB, C), I32),
            pltpu.VMEM((CPB * C,), F32),
            pltpu.VMEM((CPB * C,), F32),
            pltpu.VMEM((C, tw), F32),
            pltpu.VMEM((C, acw), F32),
            pltpu.VMEM((zb, acw), F32),
            pltpu.VMEM_SHARED((n, acw), F32),
            pltpu.SemaphoreType.DMA,
        ],
    )


def _fin_mid(p0, p1, h, r, b, degc, wnext, cout, elu, deg_from_p, bn=1000):
    n, acw = p0.shape
    cin = h.shape[1]
    mnext = wnext.shape[1]

    def body(p0_ref, p1_ref, h_ref, r_ref, b_ref, d_ref, wn_ref,
             ho_ref, yo_ref, do_ref=None):
        s = p0_ref[...] + p1_ref[...]
        agg = s[:, :cout]
        if deg_from_p:
            dg = jnp.maximum(s[:, cout:cout + 1], 1.0)
        else:
            dg = d_ref[...]
        pre = (agg / dg
               + jnp.dot(h_ref[...], r_ref[...], preferred_element_type=F32)
               + b_ref[...])
        if elu:
            pre = jnp.where(pre > 0, pre, jnp.exp(pre) - 1.0)
        ho_ref[...] = pre
        yo_ref[...] = jnp.dot(pre, wn_ref[...], preferred_element_type=F32)
        if deg_from_p:
            do_ref[...] = dg

    out_shape = [jax.ShapeDtypeStruct((n, cout), F32),
                 jax.ShapeDtypeStruct((n, mnext), F32)]
    out_specs = [pl.BlockSpec((bn, cout), lambda i: (i, 0)),
                 pl.BlockSpec((bn, mnext), lambda i: (i, 0))]
    if deg_from_p:
        out_shape.append(jax.ShapeDtypeStruct((n, 1), F32))
        out_specs.append(pl.BlockSpec((bn, 1), lambda i: (i, 0)))
    return pl.pallas_call(
        body,
        grid=(n // bn,),
        in_specs=[pl.BlockSpec((bn, acw), lambda i: (i, 0)),
                  pl.BlockSpec((bn, acw), lambda i: (i, 0)),
                  pl.BlockSpec((bn, cin), lambda i: (i, 0)),
                  pl.BlockSpec((cin, cout), lambda i: (0, 0)),
                  pl.BlockSpec((1, cout), lambda i: (0, 0)),
                  pl.BlockSpec((bn, 1), lambda i: (i, 0)),
                  pl.BlockSpec((cout, mnext), lambda i: (0, 0))],
        out_specs=out_specs,
        out_shape=out_shape,
    )(p0, p1, h, r, b, degc, wnext)


def _fin_last(p0, p1, h, r, b, degc, cout, bn=1000):
    n, acw = p0.shape
    cin = h.shape[1]

    def body(p0_ref, p1_ref, h_ref, r_ref, b_ref, d_ref, o_ref):
        s = p0_ref[...] + p1_ref[...]
        pre = (s[:, :cout] / d_ref[...]
               + jnp.dot(h_ref[...], r_ref[...], preferred_element_type=F32)
               + b_ref[...])
        m = jnp.max(pre, axis=1, keepdims=True)
        z = pre - m
        o_ref[...] = z - jnp.log(jnp.sum(jnp.exp(z), axis=1, keepdims=True))

    return pl.pallas_call(
        body,
        grid=(n // bn,),
        in_specs=[pl.BlockSpec((bn, acw), lambda i: (i, 0)),
                  pl.BlockSpec((bn, acw), lambda i: (i, 0)),
                  pl.BlockSpec((bn, cin), lambda i: (i, 0)),
                  pl.BlockSpec((cin, cout), lambda i: (0, 0)),
                  pl.BlockSpec((1, cout), lambda i: (0, 0)),
                  pl.BlockSpec((bn, 1), lambda i: (i, 0))],
        out_specs=pl.BlockSpec((bn, cout), lambda i: (i, 0)),
        out_shape=jax.ShapeDtypeStruct((n, cout), F32),
    )(p0, p1, h, r, b, degc)


def _wpair(w, cw, tw):
    k, cin, cout = w.shape
    wp = jnp.pad(w, ((0, 1), (0, 0), (0, cw - cout)))
    w0 = jnp.transpose(wp[:-1], (1, 0, 2))
    w1 = jnp.transpose(wp[1:], (1, 0, 2))
    pad = jnp.zeros((cin, k, tw - 2 * cw), F32)
    return jnp.concatenate([w0, w1, pad], axis=2).reshape(cin, k * tw)


def kernel(x, edge_index, edge_attr,
           W1, R1, B1, W2, R2, B2, W3, R3, B3,
           W4, R4, B4, W5, R5, B5, W6, R6, B6):
    n = x.shape[0]
    e = edge_index.shape[1]
    src2 = edge_index[0].reshape(e // 128, 128)
    u2 = edge_attr[:, 0].reshape(e // 128, 128)
    (g05, w05, w15, g07, w07, w17, g011, w011, w111) = _prep(src2, u2)

    dst3 = edge_index[1].reshape(e // (C * CPB), CPB, C)
    ev = lambda a: a.reshape(e)
    ev3 = lambda a: a.reshape(e // (C * CPB), CPB, C)
    eg = {5: (ev3(g05), ev(w05), ev(w15)),
          7: (ev3(g07), ev(w07), ev(w17)),
          11: (ev3(g011), ev(w011), ev(w111))}

    couts = (16, 32, 64, 128, 128, 2)
    cws = (64, 64, 64, 128, 128, 64)
    tws = (128, 128, 128, 256, 256, 128)
    acws = (128, 128, 128, 128, 128, 128)
    ks = (5, 5, 7, 7, 11, 11)
    ws = (W1, W2, W3, W4, W5, W6)
    rs = (R1, R2, R3, R4, R5, R6)
    bs = (B1, B2, B3, B4, B5, B6)
    wpairs = [_wpair(w, cw, tw) for w, cw, tw in zip(ws, cws, tws)]

    w1a = jnp.pad(W1, ((0, 0), (0, 1), (0, 1)))
    w1a = w1a.at[:, 1, 16].set(1.0)
    x_aug = jnp.concatenate([x, jnp.ones((n, 1), F32)], axis=1)
    wpairs[0] = _wpair(w1a, cws[0], tws[0])

    h = x
    degc = jnp.zeros((n, 1), F32)
    y = _matmul(x_aug, wpairs[0]).reshape(n * ks[0], tws[0])
    for li in range(6):
        k, cout = ks[li], couts[li]
        p = _edge_pass(n, e, tws[li], cws[li], acws[li])(y, *eg[k], dst3)
        b2 = bs[li].reshape(1, cout)
        if li == 0:
            h, ynext, degc = _fin_mid(p[0], p[1], h, rs[li], b2, degc,
                                      wpairs[li + 1], cout, True, True)
            y = ynext.reshape(n * ks[li + 1], tws[li + 1])
        elif li < 5:
            h, ynext = _fin_mid(p[0], p[1], h, rs[li], b2, degc,
                                wpairs[li + 1], cout, li % 2 == 0, False)
            y = ynext.reshape(n * ks[li + 1], tws[li + 1])
        else:
            h = _fin_last(p[0], p[1], h, rs[li], b2, degc, cout)
    return h


# speedup vs baseline: 5.1739x; 1.0003x over previous
import functools

import jax
import jax.numpy as jnp
from jax import lax
from jax.experimental import pallas as pl
from jax.experimental.pallas import tpu as pltpu
from jax.experimental.pallas import tpu_sc as plsc

NC = 2
NS = 16
NW = NC * NS
C = 80
CPB = 25
F32 = jnp.float32
I32 = jnp.int32


def _mm_body(h_ref, w_ref, o_ref):
    o_ref[...] = jnp.dot(h_ref[...], w_ref[...], preferred_element_type=F32)


def _matmul(h, w, bn=1000):
    n, cin = h.shape
    m = w.shape[1]
    return pl.pallas_call(
        _mm_body,
        grid=(n // bn,),
        in_specs=[pl.BlockSpec((bn, cin), lambda i: (i, 0)),
                  pl.BlockSpec((cin, m), lambda i: (0, 0))],
        out_specs=pl.BlockSpec((bn, m), lambda i: (i, 0)),
        out_shape=jax.ShapeDtypeStruct((n, m), F32),
    )(h, w)


_KS = (5, 7, 11)


def _prep_body(s_ref, u_ref, *o_refs):
    s = s_ref[...]
    u = jnp.clip(u_ref[...], 0.0, 1.0)
    i = 0
    for k in _KS:
        v = u * (k - 1.0)
        f0 = jnp.floor(v)
        i0 = jnp.clip(f0.astype(I32), 0, k - 1)
        frac = v - f0
        o_refs[i][...] = s * k + i0
        o_refs[i + 1][...] = 1.0 - frac
        o_refs[i + 2][...] = frac
        i += 3


def _prep(src2, u2, bn=1000):
    r = src2.shape[0]
    spec = pl.BlockSpec((bn, 128), lambda i: (i, 0))
    shapes = []
    for _ in _KS:
        shapes += [jax.ShapeDtypeStruct((r, 128), I32)]
        shapes += [jax.ShapeDtypeStruct((r, 128), F32)] * 2
    return pl.pallas_call(
        _prep_body,
        grid=(r // bn,),
        in_specs=[spec, spec],
        out_specs=[spec] * 9,
        out_shape=shapes,
    )(src2, u2)


@functools.lru_cache(maxsize=None)
def _edge_pass(n, e, tw, cw):
    acw = 128
    epw = e // NW
    nblk = epw // (C * CPB)
    rt = (n // NS) // 8 * 8
    tail = n - rt * NS
    zb = 16
    ncol = cw // 16
    mesh = plsc.VectorSubcoreMesh(core_axis_name="c", subcore_axis_name="s",
                                  num_cores=NC, num_subcores=NS)

    def body(t, g0v, w0, w1, dstv, out,
             g0b, dstb, w0b, w1b, rows, comb, zbuf, acc, sem):
        cid = lax.axis_index("c")
        sid = lax.axis_index("s")

        def zfill(i, _):
            for tt in range(acw // 16):
                zbuf[i, pl.ds(tt * 16, 16)] = jnp.zeros((16,), F32)
            return 0
        lax.fori_loop(0, zb, zfill, 0)
        rbase = sid * rt
        for j in range(rt // zb):
            pltpu.sync_copy(zbuf, acc.at[pl.ds(rbase + j * zb, zb)])

        @pl.when(sid == 0)
        def _():
            pltpu.sync_copy(zbuf.at[pl.ds(0, tail)],
                            acc.at[pl.ds(NS * rt, tail)])

        if cw < acw:
            def cfill(i, _):
                for tt in range(ncol, acw // 16):
                    comb[i, pl.ds(tt * 16, 16)] = jnp.zeros((16,), F32)
                return 0
            lax.fori_loop(0, C, cfill, 0)
        plsc.subcore_barrier()

        wid = cid * NS + sid
        ebase = wid * epw

        def block(nb, _):
            eb = ebase + nb * (C * CPB)
            pltpu.sync_copy(g0v.at[wid * nblk + nb], g0b)
            pltpu.sync_copy(dstv.at[wid * nblk + nb], dstb)
            pltpu.sync_copy(w0.at[pl.ds(eb, C * CPB)], w0b)
            pltpu.sync_copy(w1.at[pl.ds(eb, C * CPB)], w1b)

            def chunk(c, _):
                cb = c * C
                pltpu.async_copy(t.at[g0b.at[c]], rows, sem).wait()

                def group(g, _):
                    w0g = w0b[pl.ds(cb + g * 16, 16)]
                    w1g = w1b[pl.ds(cb + g * 16, 16)]
                    for j2 in range(16):
                        jj = g * 16 + j2
                        sel = jnp.full((16,), j2, I32)
                        a = w0g.at[sel].get(mode="promise_in_bounds")
                        b = w1g.at[sel].get(mode="promise_in_bounds")
                        for tt in range(ncol):
                            sl = pl.ds(tt * 16, 16)
                            comb[jj, sl] = (
                                rows[jj, sl] * a
                                + rows[jj, pl.ds(cw + tt * 16, 16)] * b)
                    return 0
                lax.fori_loop(0, C // 16, group, 0)
                pltpu.sync_copy(comb, acc.at[dstb.at[c]], add=True)
                return 0
            lax.fori_loop(0, CPB, chunk, 0)
            return 0
        lax.fori_loop(0, nblk, block, 0)
        plsc.subcore_barrier()

        pltpu.sync_copy(acc.at[pl.ds(rbase, rt)],
                        out.at[cid, pl.ds(rbase, rt)])

        @pl.when(sid == 0)
        def _():
            pltpu.sync_copy(acc.at[pl.ds(NS * rt, tail)],
                            out.at[cid, pl.ds(NS * rt, tail)])

    return pl.kernel(
        body,
        out_type=jax.ShapeDtypeStruct((NC, n, acw), F32),
        mesh=mesh,
        scratch_types=[
            pltpu.VMEM((CPB, C), I32),
            pltpu.VMEM((CPB, C), I32),
            pltpu.VMEM((CPB * C,), F32),
            pltpu.VMEM((CPB * C,), F32),
            pltpu.VMEM((C, tw), F32),
            pltpu.VMEM((C, 128), F32),
            pltpu.VMEM((zb, 128), F32),
            pltpu.VMEM_SHARED((n, 128), F32),
            pltpu.SemaphoreType.DMA,
        ],
    )


def _fin_mid(p0, p1, h, r, b, degc, wnext, cout, elu, deg_from_p, bn=1000):
    n, acw = p0.shape
    cin = h.shape[1]
    mnext = wnext.shape[1]

    def body(p0_ref, p1_ref, h_ref, r_ref, b_ref, d_ref, wn_ref,
             ho_ref, yo_ref, do_ref=None):
        s = p0_ref[...] + p1_ref[...]
        agg = s[:, :cout]
        if deg_from_p:
            dg = jnp.maximum(s[:, cout:cout + 1], 1.0)
        else:
            dg = d_ref[...]
        pre = (agg / dg
               + jnp.dot(h_ref[...], r_ref[...], preferred_element_type=F32)
               + b_ref[...])
        if elu:
            pre = jnp.where(pre > 0, pre, jnp.exp(pre) - 1.0)
        ho_ref[...] = pre
        yo_ref[...] = jnp.dot(pre, wn_ref[...], preferred_element_type=F32)
        if deg_from_p:
            do_ref[...] = dg

    out_shape = [jax.ShapeDtypeStruct((n, cout), F32),
                 jax.ShapeDtypeStruct((n, mnext), F32)]
    out_specs = [pl.BlockSpec((bn, cout), lambda i: (i, 0)),
                 pl.BlockSpec((bn, mnext), lambda i: (i, 0))]
    if deg_from_p:
        out_shape.append(jax.ShapeDtypeStruct((n, 1), F32))
        out_specs.append(pl.BlockSpec((bn, 1), lambda i: (i, 0)))
    return pl.pallas_call(
        body,
        grid=(n // bn,),
        in_specs=[pl.BlockSpec((bn, acw), lambda i: (i, 0)),
                  pl.BlockSpec((bn, acw), lambda i: (i, 0)),
                  pl.BlockSpec((bn, cin), lambda i: (i, 0)),
                  pl.BlockSpec((cin, cout), lambda i: (0, 0)),
                  pl.BlockSpec((1, cout), lambda i: (0, 0)),
                  pl.BlockSpec((bn, 1), lambda i: (i, 0)),
                  pl.BlockSpec((cout, mnext), lambda i: (0, 0))],
        out_specs=out_specs,
        out_shape=out_shape,
    )(p0, p1, h, r, b, degc, wnext)


def _fin_last(p0, p1, h, r, b, degc, cout, bn=1000):
    n, acw = p0.shape
    cin = h.shape[1]

    def body(p0_ref, p1_ref, h_ref, r_ref, b_ref, d_ref, o_ref):
        s = p0_ref[...] + p1_ref[...]
        pre = (s[:, :cout] / d_ref[...]
               + jnp.dot(h_ref[...], r_ref[...], preferred_element_type=F32)
               + b_ref[...])
        m = jnp.max(pre, axis=1, keepdims=True)
        z = pre - m
        o_ref[...] = z - jnp.log(jnp.sum(jnp.exp(z), axis=1, keepdims=True))

    return pl.pallas_call(
        body,
        grid=(n // bn,),
        in_specs=[pl.BlockSpec((bn, acw), lambda i: (i, 0)),
                  pl.BlockSpec((bn, acw), lambda i: (i, 0)),
                  pl.BlockSpec((bn, cin), lambda i: (i, 0)),
                  pl.BlockSpec((cin, cout), lambda i: (0, 0)),
                  pl.BlockSpec((1, cout), lambda i: (0, 0)),
                  pl.BlockSpec((bn, 1), lambda i: (i, 0))],
        out_specs=pl.BlockSpec((bn, cout), lambda i: (i, 0)),
        out_shape=jax.ShapeDtypeStruct((n, cout), F32),
    )(p0, p1, h, r, b, degc)


def _wpair(w, cw, tw):
    k, cin, cout = w.shape
    wp = jnp.pad(w, ((0, 1), (0, 0), (0, cw - cout)))
    w0 = jnp.transpose(wp[:-1], (1, 0, 2))
    w1 = jnp.transpose(wp[1:], (1, 0, 2))
    pad = jnp.zeros((cin, k, tw - 2 * cw), F32)
    return jnp.concatenate([w0, w1, pad], axis=2).reshape(cin, k * tw)


def kernel(x, edge_index, edge_attr,
           W1, R1, B1, W2, R2, B2, W3, R3, B3,
           W4, R4, B4, W5, R5, B5, W6, R6, B6):
    n = x.shape[0]
    e = edge_index.shape[1]
    src2 = edge_index[0].reshape(e // 128, 128)
    u2 = edge_attr[:, 0].reshape(e // 128, 128)
    (g05, w05, w15, g07, w07, w17, g011, w011, w111) = _prep(src2, u2)

    dst3 = edge_index[1].reshape(e // (C * CPB), CPB, C)
    ev = lambda a: a.reshape(e)
    ev3 = lambda a: a.reshape(e // (C * CPB), CPB, C)
    eg = {5: (ev3(g05), ev(w05), ev(w15)),
          7: (ev3(g07), ev(w07), ev(w17)),
          11: (ev3(g011), ev(w011), ev(w111))}

    couts = (16, 32, 64, 128, 128, 2)
    cws = (64, 64, 64, 128, 128, 64)
    tws = (128, 128, 128, 256, 256, 128)
    ks = (5, 5, 7, 7, 11, 11)
    ws = (W1, W2, W3, W4, W5, W6)
    rs = (R1, R2, R3, R4, R5, R6)
    bs = (B1, B2, B3, B4, B5, B6)
    wpairs = [_wpair(w, cw, tw) for w, cw, tw in zip(ws, cws, tws)]

    w1a = jnp.pad(W1, ((0, 0), (0, 1), (0, 1)))
    w1a = w1a.at[:, 1, 16].set(1.0)
    x_aug = jnp.concatenate([x, jnp.ones((n, 1), F32)], axis=1)
    wpairs[0] = _wpair(w1a, cws[0], tws[0])

    h = x
    degc = jnp.zeros((n, 1), F32)
    y = _matmul(x_aug, wpairs[0]).reshape(n * ks[0], tws[0])
    for li in range(6):
        k, cout = ks[li], couts[li]
        p = _edge_pass(n, e, tws[li], cws[li])(y, *eg[k], dst3)
        b2 = bs[li].reshape(1, cout)
        if li == 0:
            h, ynext, degc = _fin_mid(p[0], p[1], h, rs[li], b2, degc,
                                      wpairs[li + 1], cout, True, True)
            y = ynext.reshape(n * ks[li + 1], tws[li + 1])
        elif li < 5:
            h, ynext = _fin_mid(p[0], p[1], h, rs[li], b2, degc,
                                wpairs[li + 1], cout, li % 2 == 0, False)
            y = ynext.reshape(n * ks[li + 1], tws[li + 1])
        else:
            h = _fin_last(p[0], p[1], h, rs[li], b2, degc, cout)
    return h


# speedup vs baseline: 5.9895x; 1.1576x over previous
import functools

import jax
import jax.numpy as jnp
from jax import lax
from jax.experimental import pallas as pl
from jax.experimental.pallas import tpu as pltpu
from jax.experimental.pallas import tpu_sc as plsc

NC = 2
NS = 16
NW = NC * NS
C = 80
CPB = 25
F32 = jnp.float32
I32 = jnp.int32


def _mm_body(h_ref, w_ref, o_ref):
    o_ref[...] = jnp.dot(h_ref[...], w_ref[...], preferred_element_type=F32)


def _matmul(h, w, bn=1000):
    n, cin = h.shape
    m = w.shape[1]
    return pl.pallas_call(
        _mm_body,
        grid=(n // bn,),
        in_specs=[pl.BlockSpec((bn, cin), lambda i: (i, 0)),
                  pl.BlockSpec((cin, m), lambda i: (0, 0))],
        out_specs=pl.BlockSpec((bn, m), lambda i: (i, 0)),
        out_shape=jax.ShapeDtypeStruct((n, m), F32),
    )(h, w)


_KS = (5, 7, 11)


def _prep_body(s_ref, u_ref, *o_refs):
    s = s_ref[...]
    u = jnp.clip(u_ref[...], 0.0, 1.0)
    i = 0
    for k in _KS:
        v = u * (k - 1.0)
        f0 = jnp.floor(v)
        i0 = jnp.clip(f0.astype(I32), 0, k - 1)
        frac = v - f0
        o_refs[i][...] = s * k + i0
        o_refs[i + 1][...] = 1.0 - frac
        o_refs[i + 2][...] = frac
        i += 3


def _prep(src2, u2, bn=1000):
    r = src2.shape[0]
    spec = pl.BlockSpec((bn, 128), lambda i: (i, 0))
    shapes = []
    for _ in _KS:
        shapes += [jax.ShapeDtypeStruct((r, 128), I32)]
        shapes += [jax.ShapeDtypeStruct((r, 128), F32)] * 2
    return pl.pallas_call(
        _prep_body,
        grid=(r // bn,),
        in_specs=[spec, spec],
        out_specs=[spec] * 9,
        out_shape=shapes,
    )(src2, u2)


@functools.lru_cache(maxsize=None)
def _edge_pass(n, e, tw, cw):
    acw = 128
    epw = e // NW
    nblk = epw // (C * CPB)
    rt = (n // NS) // 8 * 8
    tail = n - rt * NS
    zb = 16
    ncol = cw // 16
    dbl = tw == 128
    rb2 = C if dbl else 8
    mesh = plsc.VectorSubcoreMesh(core_axis_name="c", subcore_axis_name="s",
                                  num_cores=NC, num_subcores=NS)

    def body(t, g0v, w0, w1, dstv, out,
             g0b, dstb, w0b, w1b, rows, rows2, comb, zbuf, acc, sem, sem2):
        cid = lax.axis_index("c")
        sid = lax.axis_index("s")

        def zfill(i, _):
            for tt in range(acw // 16):
                zbuf[i, pl.ds(tt * 16, 16)] = jnp.zeros((16,), F32)
            return 0
        lax.fori_loop(0, zb, zfill, 0)
        rbase = sid * rt
        for j in range(rt // zb):
            pltpu.sync_copy(zbuf, acc.at[pl.ds(rbase + j * zb, zb)])

        @pl.when(sid == 0)
        def _():
            pltpu.sync_copy(zbuf.at[pl.ds(0, tail)],
                            acc.at[pl.ds(NS * rt, tail)])

        if cw < acw:
            def cfill(i, _):
                for tt in range(ncol, acw // 16):
                    comb[i, pl.ds(tt * 16, 16)] = jnp.zeros((16,), F32)
                return 0
            lax.fori_loop(0, C, cfill, 0)
        plsc.subcore_barrier()

        wid = cid * NS + sid
        ebase = wid * epw

        def block(nb, _):
            eb = ebase + nb * (C * CPB)
            pltpu.sync_copy(g0v.at[wid * nblk + nb], g0b)
            pltpu.sync_copy(dstv.at[wid * nblk + nb], dstb)
            pltpu.sync_copy(w0.at[pl.ds(eb, C * CPB)], w0b)
            pltpu.sync_copy(w1.at[pl.ds(eb, C * CPB)], w1b)

            def work(buf, cb, c):
                def group(g, _):
                    w0g = w0b[pl.ds(cb + g * 16, 16)]
                    w1g = w1b[pl.ds(cb + g * 16, 16)]
                    for j2 in range(16):
                        jj = g * 16 + j2
                        sel = jnp.full((16,), j2, I32)
                        a = w0g.at[sel].get(mode="promise_in_bounds")
                        b = w1g.at[sel].get(mode="promise_in_bounds")
                        for tt in range(ncol):
                            sl = pl.ds(tt * 16, 16)
                            comb[jj, sl] = (
                                buf[jj, sl] * a
                                + buf[jj, pl.ds(cw + tt * 16, 16)] * b)
                    return 0
                lax.fori_loop(0, C // 16, group, 0)
                pltpu.sync_copy(comb, acc.at[dstb.at[c]], add=True)

            if not dbl:
                def chunk(c, _):
                    pltpu.async_copy(t.at[g0b.at[c]], rows, sem).wait()
                    work(rows, c * C, c)
                    return 0
                lax.fori_loop(0, CPB, chunk, 0)
            else:
                pltpu.async_copy(t.at[g0b.at[0]], rows, sem)

                def chunk(c, _):
                    @pl.when(c % 2 == 0)
                    def _():
                        pltpu.make_async_copy(t.at[g0b.at[c]], rows,
                                              sem).wait()

                        @pl.when(c + 1 < CPB)
                        def _():
                            pltpu.async_copy(t.at[g0b.at[c + 1]], rows2,
                                             sem2)
                        work(rows, c * C, c)

                    @pl.when(c % 2 == 1)
                    def _():
                        pltpu.make_async_copy(t.at[g0b.at[c]], rows2,
                                              sem2).wait()

                        @pl.when(c + 1 < CPB)
                        def _():
                            pltpu.async_copy(t.at[g0b.at[c + 1]], rows,
                                             sem)
                        work(rows2, c * C, c)
                    return 0
                lax.fori_loop(0, CPB, chunk, 0)
            return 0
        lax.fori_loop(0, nblk, block, 0)
        plsc.subcore_barrier()

        pltpu.sync_copy(acc.at[pl.ds(rbase, rt)],
                        out.at[cid, pl.ds(rbase, rt)])

        @pl.when(sid == 0)
        def _():
            pltpu.sync_copy(acc.at[pl.ds(NS * rt, tail)],
                            out.at[cid, pl.ds(NS * rt, tail)])

    return pl.kernel(
        body,
        out_type=jax.ShapeDtypeStruct((NC, n, acw), F32),
        mesh=mesh,
        scratch_types=[
            pltpu.VMEM((CPB, C), I32),
            pltpu.VMEM((CPB, C), I32),
            pltpu.VMEM((CPB * C,), F32),
            pltpu.VMEM((CPB * C,), F32),
            pltpu.VMEM((C, tw), F32),
            pltpu.VMEM((rb2, tw), F32),
            pltpu.VMEM((C, 128), F32),
            pltpu.VMEM((zb, 128), F32),
            pltpu.VMEM_SHARED((n, 128), F32),
            pltpu.SemaphoreType.DMA,
            pltpu.SemaphoreType.DMA,
        ],
    )


def _fin_mid(p0, p1, h, r, b, degc, wnext, cout, elu, deg_from_p, bn=1000):
    n, acw = p0.shape
    cin = h.shape[1]
    mnext = wnext.shape[1]

    def body(p0_ref, p1_ref, h_ref, r_ref, b_ref, d_ref, wn_ref,
             ho_ref, yo_ref, do_ref=None):
        s = p0_ref[...] + p1_ref[...]
        agg = s[:, :cout]
        if deg_from_p:
            dg = jnp.maximum(s[:, cout:cout + 1], 1.0)
        else:
            dg = d_ref[...]
        pre = (agg / dg
               + jnp.dot(h_ref[...], r_ref[...], preferred_element_type=F32)
               + b_ref[...])
        if elu:
            pre = jnp.where(pre > 0, pre, jnp.exp(pre) - 1.0)
        ho_ref[...] = pre
        yo_ref[...] = jnp.dot(pre, wn_ref[...], preferred_element_type=F32)
        if deg_from_p:
            do_ref[...] = dg

    out_shape = [jax.ShapeDtypeStruct((n, cout), F32),
                 jax.ShapeDtypeStruct((n, mnext), F32)]
    out_specs = [pl.BlockSpec((bn, cout), lambda i: (i, 0)),
                 pl.BlockSpec((bn, mnext), lambda i: (i, 0))]
    if deg_from_p:
        out_shape.append(jax.ShapeDtypeStruct((n, 1), F32))
        out_specs.append(pl.BlockSpec((bn, 1), lambda i: (i, 0)))
    return pl.pallas_call(
        body,
        grid=(n // bn,),
        in_specs=[pl.BlockSpec((bn, acw), lambda i: (i, 0)),
                  pl.BlockSpec((bn, acw), lambda i: (i, 0)),
                  pl.BlockSpec((bn, cin), lambda i: (i, 0)),
                  pl.BlockSpec((cin, cout), lambda i: (0, 0)),
                  pl.BlockSpec((1, cout), lambda i: (0, 0)),
                  pl.BlockSpec((bn, 1), lambda i: (i, 0)),
                  pl.BlockSpec((cout, mnext), lambda i: (0, 0))],
        out_specs=out_specs,
        out_shape=out_shape,
    )(p0, p1, h, r, b, degc, wnext)


def _fin_last(p0, p1, h, r, b, degc, cout, bn=1000):
    n, acw = p0.shape
    cin = h.shape[1]

    def body(p0_ref, p1_ref, h_ref, r_ref, b_ref, d_ref, o_ref):
        s = p0_ref[...] + p1_ref[...]
        pre = (s[:, :cout] / d_ref[...]
               + jnp.dot(h_ref[...], r_ref[...], preferred_element_type=F32)
               + b_ref[...])
        m = jnp.max(pre, axis=1, keepdims=True)
        z = pre - m
        o_ref[...] = z - jnp.log(jnp.sum(jnp.exp(z), axis=1, keepdims=True))

    return pl.pallas_call(
        body,
        grid=(n // bn,),
        in_specs=[pl.BlockSpec((bn, acw), lambda i: (i, 0)),
                  pl.BlockSpec((bn, acw), lambda i: (i, 0)),
                  pl.BlockSpec((bn, cin), lambda i: (i, 0)),
                  pl.BlockSpec((cin, cout), lambda i: (0, 0)),
                  pl.BlockSpec((1, cout), lambda i: (0, 0)),
                  pl.BlockSpec((bn, 1), lambda i: (i, 0))],
        out_specs=pl.BlockSpec((bn, cout), lambda i: (i, 0)),
        out_shape=jax.ShapeDtypeStruct((n, cout), F32),
    )(p0, p1, h, r, b, degc)


def _wpair(w, cw, tw):
    k, cin, cout = w.shape
    wp = jnp.pad(w, ((0, 1), (0, 0), (0, cw - cout)))
    w0 = jnp.transpose(wp[:-1], (1, 0, 2))
    w1 = jnp.transpose(wp[1:], (1, 0, 2))
    pad = jnp.zeros((cin, k, tw - 2 * cw), F32)
    return jnp.concatenate([w0, w1, pad], axis=2).reshape(cin, k * tw)


def kernel(x, edge_index, edge_attr,
           W1, R1, B1, W2, R2, B2, W3, R3, B3,
           W4, R4, B4, W5, R5, B5, W6, R6, B6):
    n = x.shape[0]
    e = edge_index.shape[1]
    src2 = edge_index[0].reshape(e // 128, 128)
    u2 = edge_attr[:, 0].reshape(e // 128, 128)
    (g05, w05, w15, g07, w07, w17, g011, w011, w111) = _prep(src2, u2)

    dst3 = edge_index[1].reshape(e // (C * CPB), CPB, C)
    ev = lambda a: a.reshape(e)
    ev3 = lambda a: a.reshape(e // (C * CPB), CPB, C)
    eg = {5: (ev3(g05), ev(w05), ev(w15)),
          7: (ev3(g07), ev(w07), ev(w17)),
          11: (ev3(g011), ev(w011), ev(w111))}

    couts = (16, 32, 64, 128, 128, 2)
    cws = (64, 64, 64, 128, 128, 64)
    tws = (128, 128, 128, 256, 256, 128)
    ks = (5, 5, 7, 7, 11, 11)
    ws = (W1, W2, W3, W4, W5, W6)
    rs = (R1, R2, R3, R4, R5, R6)
    bs = (B1, B2, B3, B4, B5, B6)
    wpairs = [_wpair(w, cw, tw) for w, cw, tw in zip(ws, cws, tws)]

    w1a = jnp.pad(W1, ((0, 0), (0, 1), (0, 1)))
    w1a = w1a.at[:, 1, 16].set(1.0)
    x_aug = jnp.concatenate([x, jnp.ones((n, 1), F32)], axis=1)
    wpairs[0] = _wpair(w1a, cws[0], tws[0])

    h = x
    degc = jnp.zeros((n, 1), F32)
    y = _matmul(x_aug, wpairs[0]).reshape(n * ks[0], tws[0])
    for li in range(6):
        k, cout = ks[li], couts[li]
        p = _edge_pass(n, e, tws[li], cws[li])(y, *eg[k], dst3)
        b2 = bs[li].reshape(1, cout)
        if li == 0:
            h, ynext, degc = _fin_mid(p[0], p[1], h, rs[li], b2, degc,
                                      wpairs[li + 1], cout, True, True)
            y = ynext.reshape(n * ks[li + 1], tws[li + 1])
        elif li < 5:
            h, ynext = _fin_mid(p[0], p[1], h, rs[li], b2, degc,
                                wpairs[li + 1], cout, li % 2 == 0, False)
            y = ynext.reshape(n * ks[li + 1], tws[li + 1])
        else:
            h = _fin_last(p[0], p[1], h, rs[li], b2, degc, cout)
    return h
